# baseline (device time: 31960 ns/iter reference)
import math

import jax
import jax.numpy as jnp
from jax import lax
from jax.experimental import pallas as pl
from jax.experimental.pallas import tpu as pltpu

N_DEV = 16

_SEND_OFFSETS = [1, 15, 4, 12, 2, 14, 3, 13, 5, 11, 8, 6, 10, 7, 9]
_CHUNKS = [[15, 1, 12, 4], [14, 2, 13, 3], [11, 5, 8, 10], [6, 9], [7]]


def kernel(q, k, v):
    s_per, d = q.shape
    scale = 1.0 / math.sqrt(d)

    def body(q_ref, k_ref, v_ref, out_ref, mine_ref, comm_ref,
             send_sems, recv_sems, ready_sems):
        my = lax.axis_index("i")

        barrier = pltpu.get_barrier_semaphore()
        pl.semaphore_signal(
            barrier, inc=1,
            device_id=(my,), device_id_type=pl.DeviceIdType.MESH,
        )
        pl.semaphore_wait(barrier, 1)

        for o in _SEND_OFFSETS:
            pl.semaphore_signal(
                ready_sems.at[my], inc=1,
                device_id=(lax.rem(my + o, N_DEV),),
                device_id_type=pl.DeviceIdType.MESH,
            )

        mine_ref[0] = k_ref[...].astype(jnp.bfloat16)
        mine_ref[1] = v_ref[...].astype(jnp.bfloat16)

        sends = []

        def issue_sends(offsets):
            for o in offsets:
                tgt = lax.rem(my + o, N_DEV)
                pl.semaphore_wait(ready_sems.at[tgt], 1)
                rdma = pltpu.make_async_remote_copy(
                    src_ref=mine_ref,
                    dst_ref=comm_ref.at[my],
                    send_sem=send_sems.at[o],
                    recv_sem=recv_sems.at[my],
                    device_id=(tgt,),
                    device_id_type=pl.DeviceIdType.MESH,
                )
                rdma.start()
                sends.append(rdma)

        issue_sends(_SEND_OFFSETS[:4])

        def fold(state, kv_blks):
            m, l, acc = state
            s = jnp.concatenate(
                [
                    lax.dot_general(
                        q_blk, k_blk,
                        dimension_numbers=(((1,), (1,)), ((), ())),
                        preferred_element_type=jnp.float32,
                    )
                    for k_blk, _ in kv_blks
                ],
                axis=1,
            ) * scale
            m_new = jnp.maximum(m, jnp.max(s, axis=1, keepdims=True))
            p = jnp.exp(s - m_new)
            alpha = jnp.exp(m - m_new)
            l = l * alpha + jnp.sum(p, axis=1, keepdims=True)
            pv = acc * alpha
            for idx, (_, v_blk) in enumerate(kv_blks):
                pv = pv + lax.dot_general(
                    p[:, idx * s_per:(idx + 1) * s_per].astype(jnp.bfloat16),
                    v_blk,
                    dimension_numbers=(((1,), (0,)), ((), ())),
                    preferred_element_type=jnp.float32,
                )
            return m_new, l, pv

        q_blk = q_ref[...].astype(jnp.bfloat16)
        state = (
            jnp.full((s_per, 1), -1e30, jnp.float32),
            jnp.zeros((s_per, 1), jnp.float32),
            jnp.zeros((s_per, d), jnp.float32),
        )
        state = fold(state, [(mine_ref[0], mine_ref[1])])

        issue_sends(_SEND_OFFSETS[4:])

        for chunk in _CHUNKS:
            kv_blks = []
            for o in chunk:
                origin = lax.rem(my + o, N_DEV)
                recv = pltpu.make_async_remote_copy(
                    src_ref=mine_ref,
                    dst_ref=comm_ref.at[origin],
                    send_sem=send_sems.at[o],
                    recv_sem=recv_sems.at[origin],
                    device_id=(origin,),
                    device_id_type=pl.DeviceIdType.MESH,
                )
                recv.wait_recv()
                kv_blks.append((comm_ref[origin, 0], comm_ref[origin, 1]))
            state = fold(state, kv_blks)

        _, l, acc = state
        out_ref[...] = acc / l

        for rdma in sends:
            rdma.wait_send()

    return pl.pallas_call(
        body,
        out_shape=jax.ShapeDtypeStruct((s_per, d), jnp.float32),
        in_specs=[pl.BlockSpec(memory_space=pltpu.VMEM)] * 3,
        out_specs=pl.BlockSpec(memory_space=pltpu.VMEM),
        scratch_shapes=[
            pltpu.VMEM((2, s_per, d), jnp.bfloat16),
            pltpu.VMEM((N_DEV, 2, s_per, d), jnp.bfloat16),
            pltpu.SemaphoreType.DMA((N_DEV,)),
            pltpu.SemaphoreType.DMA((N_DEV,)),
            pltpu.SemaphoreType.REGULAR((N_DEV,)),
        ],
        compiler_params=pltpu.CompilerParams(collective_id=0),
    )(q, k, v)
